# baseline (speedup 1.0000x reference)
import functools

import jax
import jax.numpy as jnp
from jax import lax
from jax.experimental import pallas as pl
from jax.experimental.pallas import tpu as pltpu
from jax.experimental.pallas import tpu_sc as plsc

NC, NS, L = 2, 16, 16
NW = NC * NS
BATCH, SEQ = 16384, 200
N = BATCH * SEQ
CTILES = BATCH // 128
TROWS = SEQ // 8
TPB = 16
UNIT = TPB * 1024
NUNITS = TROWS * (CTILES // TPB)
FULL_PER_W = 6
QTILES = 4
QUNIT = QTILES * 1024

_mesh = plsc.VectorSubcoreMesh(
    core_axis_name="c", subcore_axis_name="s", num_cores=NC, num_subcores=NS)


@functools.partial(
    pl.kernel,
    out_type=jax.ShapeDtypeStruct((SEQ, CTILES, 128), jnp.float32),
    mesh=_mesh,
    compiler_params=pltpu.CompilerParams(needs_layout_passes=False),
    scratch_types=[
        pltpu.VMEM((32,), jnp.float32),
        pltpu.VMEM((16,), jnp.float32),
        pltpu.VMEM((16,), jnp.float32),
        pltpu.VMEM((UNIT,), jnp.int32),
        pltpu.VMEM((UNIT,), jnp.int32),
        pltpu.VMEM((8, TPB, 128), jnp.float32),
        pltpu.VMEM((8, TPB, 128), jnp.float32),
        pltpu.SemaphoreType.DMA,
        pltpu.SemaphoreType.DMA,
        pltpu.SemaphoreType.DMA,
        pltpu.SemaphoreType.DMA,
    ],
)
def _sc_lookup(x_hbm, tab_hbm, wb_hbm, out_hbm, tab_v, wb_v, lut_v, xin0,
               xin1, yout0, yout1, in_sem0, in_sem1, out_sem0, out_sem1):
    wid = lax.axis_index("s") * NC + lax.axis_index("c")
    xin_b = (xin0, xin1)
    yout_b = (yout0, yout1)
    in_sems = (in_sem0, in_sem1)
    out_sems = (out_sem0, out_sem1)

    pltpu.sync_copy(tab_hbm, tab_v)
    pltpu.sync_copy(wb_hbm, wb_v)

    lane = lax.iota(jnp.int32, 16)
    rowmask = lane < 7
    zero16 = jnp.zeros((16,), jnp.int32)

    def col(j):
        idx = jnp.where(rowmask, lane * 3 + j, 0)
        return plsc.load_gather(tab_v, [idx])

    def scalar_bcast(j):
        return plsc.load_gather(wb_v, [zero16 + (j + 1)])

    z = (col(0) * scalar_bcast(0)
         + col(1) * scalar_bcast(1)
         + col(2) * scalar_bcast(2)
         + scalar_bcast(3))
    lutv = 1.0 / (1.0 + jnp.exp(-z))
    lut_v[...] = jnp.where(rowmask, lutv, 0.0)

    q = lax.rem(wid, 4)
    tail_m = lax.div(wid, 4)

    def unit_coords(slot):
        if slot < FULL_PER_W:
            u = wid + NW * slot
            tr = lax.div(u, 8)
            m = lax.rem(u, 8)
            words = UNIT
            in_off = (tr * CTILES + m * TPB) * 1024
        else:
            tr = TROWS - 1
            m = tail_m
            words = QUNIT
            in_off = (tr * CTILES + m * TPB) * 1024 + q * QUNIT
        return tr, m, words, in_off

    def start_in(slot):
        _, _, words, in_off = unit_coords(slot)
        off = pl.multiple_of(in_off, 8)
        return pltpu.async_copy(
            x_hbm.at[pl.ds(off, words)],
            xin_b[slot % 2].at[pl.ds(0, words)], in_sems[slot % 2])

    def start_out(slot):
        tr, m, words, _ = unit_coords(slot)
        yout = yout_b[slot % 2]
        ntiles = words // 1024
        t0 = m * TPB + (q * QTILES if slot >= FULL_PER_W else 0)
        return pltpu.async_copy(
            yout.at[pl.ds(0, 8), pl.ds(0, ntiles), pl.ds(0, 128)],
            out_hbm.at[pl.ds(8 * tr, 8),
                       pl.ds(pl.multiple_of(t0, 4), ntiles), pl.ds(0, 128)],
            out_sems[slot % 2])

    def compute(slot):
        xin = xin_b[slot % 2]
        yout = yout_b[slot % 2]
        ntiles = TPB if slot < FULL_PER_W else QTILES

        @plsc.parallel_loop(0, ntiles * 8, 1, unroll=2)
        def _(vv):
            t = lax.shift_right_logical(vv, 3)
            v = lax.bitwise_and(vv, 7)
            base_w = t * 1024 + v * 16
            for ri in range(8):
                yout[ri, t, pl.ds(v * 16, L)] = plsc.load_gather(
                    lut_v, [xin[pl.ds(base_w + ri * 128, L)]])

    NSLOTS = FULL_PER_W + 1
    cp_in = [None] * NSLOTS
    cp_out = [None] * NSLOTS
    cp_in[0] = start_in(0)
    for s in range(NSLOTS):
        if s + 1 < NSLOTS:
            cp_in[s + 1] = start_in(s + 1)
        cp_in[s].wait()
        if s >= 2:
            cp_out[s - 2].wait()
        compute(s)
        cp_out[s] = start_out(s)
    cp_out[NSLOTS - 2].wait()
    cp_out[NSLOTS - 1].wait()


def kernel(x, table, W, b):
    tab = jnp.concatenate([table.reshape(-1), jnp.zeros((11,), jnp.float32)])
    wb = jnp.concatenate(
        [jnp.zeros((1,), jnp.float32), W.reshape(-1), b.reshape(-1),
         jnp.zeros((11,), jnp.float32)])
    xf = (x.T.reshape(SEQ // 8, 8, BATCH // 128, 128)
          .transpose(0, 2, 1, 3).reshape(-1))
    out = _sc_lookup(xf, tab, wb)
    return out.transpose(1, 2, 0).reshape(BATCH, SEQ)[:, :, None]


# speedup vs baseline: 1.0251x; 1.0251x over previous
import functools

import jax
import jax.numpy as jnp
from jax import lax
from jax.experimental import pallas as pl
from jax.experimental.pallas import tpu as pltpu
from jax.experimental.pallas import tpu_sc as plsc

NC, NS, L = 2, 16, 16
NW = NC * NS
BATCH, SEQ = 16384, 200
N = BATCH * SEQ
CTILES = BATCH // 128
TROWS = SEQ // 8
TPB = 16
UNIT = TPB * 1024
NUNITS = TROWS * (CTILES // TPB)
FULL_PER_W = 6
QTILES = 4
QUNIT = QTILES * 1024

_mesh = plsc.VectorSubcoreMesh(
    core_axis_name="c", subcore_axis_name="s", num_cores=NC, num_subcores=NS)


@functools.partial(
    pl.kernel,
    out_type=jax.ShapeDtypeStruct((SEQ, CTILES, 128), jnp.float32),
    mesh=_mesh,
    compiler_params=pltpu.CompilerParams(needs_layout_passes=False),
    scratch_types=[
        pltpu.VMEM((32,), jnp.float32),
        pltpu.VMEM((16,), jnp.float32),
        pltpu.VMEM((16,), jnp.float32),
        pltpu.VMEM((UNIT,), jnp.int32),
        pltpu.VMEM((UNIT,), jnp.int32),
        pltpu.VMEM((8, TPB, 128), jnp.float32),
        pltpu.VMEM((8, TPB, 128), jnp.float32),
        pltpu.SemaphoreType.DMA,
        pltpu.SemaphoreType.DMA,
        pltpu.SemaphoreType.DMA,
        pltpu.SemaphoreType.DMA,
    ],
)
def _sc_lookup(x_hbm, tab_hbm, wb_hbm, out_hbm, tab_v, wb_v, lut_v, xin0,
               xin1, yout0, yout1, in_sem0, in_sem1, out_sem0, out_sem1):
    wid = lax.axis_index("s") * NC + lax.axis_index("c")
    xin_b = (xin0, xin1)
    yout_b = (yout0, yout1)
    in_sems = (in_sem0, in_sem1)
    out_sems = (out_sem0, out_sem1)

    pltpu.sync_copy(tab_hbm, tab_v)
    pltpu.sync_copy(wb_hbm, wb_v)

    lane = lax.iota(jnp.int32, 16)
    rowmask = lane < 7
    zero16 = jnp.zeros((16,), jnp.int32)

    def col(j):
        idx = jnp.where(rowmask, lane * 3 + j, 0)
        return plsc.load_gather(tab_v, [idx])

    def scalar_bcast(j):
        return plsc.load_gather(wb_v, [zero16 + (j + 1)])

    z = (col(0) * scalar_bcast(0)
         + col(1) * scalar_bcast(1)
         + col(2) * scalar_bcast(2)
         + scalar_bcast(3))
    lutv = 1.0 / (1.0 + jnp.exp(-z))
    lut_v[...] = jnp.where(rowmask, lutv, 0.0)

    q = lax.rem(wid, 4)
    tail_m = lax.div(wid, 4)

    def unit_coords(slot):
        if slot < FULL_PER_W:
            u = wid + NW * slot
            tr = lax.div(u, 8)
            m = lax.rem(u, 8)
            words = UNIT
            in_off = (tr * CTILES + m * TPB) * 1024
        else:
            tr = TROWS - 1
            m = tail_m
            words = QUNIT
            in_off = (tr * CTILES + m * TPB) * 1024 + q * QUNIT
        return tr, m, words, in_off

    def start_in(slot):
        _, _, words, in_off = unit_coords(slot)
        off = pl.multiple_of(in_off, 8)
        return pltpu.async_copy(
            x_hbm.at[pl.ds(off, words)],
            xin_b[slot % 2].at[pl.ds(0, words)], in_sems[slot % 2])

    def start_out(slot):
        tr, m, words, _ = unit_coords(slot)
        yout = yout_b[slot % 2]
        ntiles = words // 1024
        t0 = m * TPB + (q * QTILES if slot >= FULL_PER_W else 0)
        return pltpu.async_copy(
            yout.at[pl.ds(0, 8), pl.ds(0, ntiles), pl.ds(0, 128)],
            out_hbm.at[pl.ds(8 * tr, 8),
                       pl.ds(pl.multiple_of(t0, 4), ntiles), pl.ds(0, 128)],
            out_sems[slot % 2])

    def compute(slot):
        xin = xin_b[slot % 2]
        yout = yout_b[slot % 2]
        ntiles = TPB if slot < FULL_PER_W else QTILES

        @plsc.parallel_loop(0, ntiles * 8, 1)
        def _(vv):
            t = lax.shift_right_logical(vv, 3)
            v = lax.bitwise_and(vv, 7)
            base_w = t * 1024 + v * 16
            for ri in range(8):
                yout[ri, t, pl.ds(v * 16, L)] = plsc.load_gather(
                    lut_v, [xin[pl.ds(base_w + ri * 128, L)]])

    NSLOTS = FULL_PER_W + 1
    cp_in = [None] * NSLOTS
    cp_out = [None] * NSLOTS
    cp_in[0] = start_in(0)
    for s in range(NSLOTS):
        if s + 1 < NSLOTS:
            cp_in[s + 1] = start_in(s + 1)
        cp_in[s].wait()
        if s >= 2:
            cp_out[s - 2].wait()
        compute(s)
        cp_out[s] = start_out(s)
    cp_out[NSLOTS - 2].wait()
    cp_out[NSLOTS - 1].wait()


def kernel(x, table, W, b):
    tab = jnp.concatenate([table.reshape(-1), jnp.zeros((11,), jnp.float32)])
    wb = jnp.concatenate(
        [jnp.zeros((1,), jnp.float32), W.reshape(-1), b.reshape(-1),
         jnp.zeros((11,), jnp.float32)])
    xf = (x.T.reshape(SEQ // 8, 8, BATCH // 128, 128)
          .transpose(0, 2, 1, 3).reshape(-1))
    out = _sc_lookup(xf, tab, wb)
    return out.transpose(1, 2, 0).reshape(BATCH, SEQ)[:, :, None]


# speedup vs baseline: 1.0718x; 1.0455x over previous
import functools

import jax
import jax.numpy as jnp
from jax import lax
from jax.experimental import pallas as pl
from jax.experimental.pallas import tpu as pltpu
from jax.experimental.pallas import tpu_sc as plsc

NC, NS, L = 2, 16, 16
NW = NC * NS
BATCH, SEQ = 16384, 200
N = BATCH * SEQ
CTILES = BATCH // 128
TROWS = SEQ // 8
TPB = 16
UNIT = TPB * 1024
NUNITS = TROWS * (CTILES // TPB)
FULL_PER_W = 6
QTILES = 4
QUNIT = QTILES * 1024

_mesh = plsc.VectorSubcoreMesh(
    core_axis_name="c", subcore_axis_name="s", num_cores=NC, num_subcores=NS)


@functools.partial(
    pl.kernel,
    out_type=jax.ShapeDtypeStruct((SEQ, CTILES, 128), jnp.float32),
    mesh=_mesh,
    compiler_params=pltpu.CompilerParams(needs_layout_passes=False),
    scratch_types=[
        pltpu.VMEM((32,), jnp.float32),
        pltpu.VMEM((16,), jnp.float32),
        pltpu.VMEM((16,), jnp.float32),
        pltpu.VMEM((UNIT,), jnp.int32),
        pltpu.VMEM((UNIT,), jnp.int32),
        pltpu.VMEM((8, TPB, 128), jnp.float32),
        pltpu.VMEM((8, TPB, 128), jnp.float32),
        pltpu.SemaphoreType.DMA,
        pltpu.SemaphoreType.DMA,
        pltpu.SemaphoreType.DMA,
        pltpu.SemaphoreType.DMA,
    ],
)
def _sc_lookup(x_hbm, tab_hbm, wb_hbm, out_hbm, tab_v, wb_v, lut_v, xin0,
               xin1, yout0, yout1, in_sem0, in_sem1, out_sem0, out_sem1):
    wid = lax.axis_index("s") * NC + lax.axis_index("c")
    xin_b = (xin0, xin1)
    yout_b = (yout0, yout1)
    in_sems = (in_sem0, in_sem1)
    out_sems = (out_sem0, out_sem1)

    lane = lax.iota(jnp.int32, 16)
    rowmask = lane < 7
    zero16 = jnp.zeros((16,), jnp.int32)

    def col(j):
        idx = jnp.where(rowmask, lane * 3 + j, 0)
        return plsc.load_gather(tab_v, [idx])

    def scalar_bcast(j):
        return plsc.load_gather(wb_v, [zero16 + (j + 1)])

    def make_lut():
        pltpu.sync_copy(tab_hbm, tab_v)
        pltpu.sync_copy(wb_hbm, wb_v)
        z = (col(0) * scalar_bcast(0)
             + col(1) * scalar_bcast(1)
             + col(2) * scalar_bcast(2)
             + scalar_bcast(3))
        lutv = 1.0 / (1.0 + jnp.exp(-z))
        lut_v[...] = jnp.where(rowmask, lutv, 0.0)

    q = lax.rem(wid, 4)
    tail_m = lax.div(wid, 4)

    def unit_coords(slot):
        if slot < FULL_PER_W:
            u = wid + NW * slot
            tr = lax.div(u, 8)
            m = lax.rem(u, 8)
            words = UNIT
            in_off = (tr * CTILES + m * TPB) * 1024
        else:
            tr = TROWS - 1
            m = tail_m
            words = QUNIT
            in_off = (tr * CTILES + m * TPB) * 1024 + q * QUNIT
        return tr, m, words, in_off

    def start_in(slot):
        _, _, words, in_off = unit_coords(slot)
        off = pl.multiple_of(in_off, 8)
        return pltpu.async_copy(
            x_hbm.at[pl.ds(off, words)],
            xin_b[slot % 2].at[pl.ds(0, words)], in_sems[slot % 2])

    def start_out(slot):
        tr, m, words, _ = unit_coords(slot)
        yout = yout_b[slot % 2]
        ntiles = words // 1024
        t0 = m * TPB + (q * QTILES if slot >= FULL_PER_W else 0)
        return pltpu.async_copy(
            yout.at[pl.ds(0, 8), pl.ds(0, ntiles), pl.ds(0, 128)],
            out_hbm.at[pl.ds(8 * tr, 8),
                       pl.ds(pl.multiple_of(t0, 4), ntiles), pl.ds(0, 128)],
            out_sems[slot % 2])

    def compute(slot):
        xin = xin_b[slot % 2]
        yout = yout_b[slot % 2]
        ntiles = TPB if slot < FULL_PER_W else QTILES

        @plsc.parallel_loop(0, ntiles * 8, 1)
        def _(vv):
            t = lax.shift_right_logical(vv, 3)
            v = lax.bitwise_and(vv, 7)
            base_w = t * 1024 + v * 16
            for ri in range(8):
                yout[ri, t, pl.ds(v * 16, L)] = plsc.load_gather(
                    lut_v, [xin[pl.ds(base_w + ri * 128, L)]])

    NSLOTS = FULL_PER_W + 1
    cp_in = [None] * NSLOTS
    cp_out = [None] * NSLOTS
    cp_in[0] = start_in(0)
    make_lut()
    for s in range(NSLOTS):
        if s + 1 < NSLOTS:
            cp_in[s + 1] = start_in(s + 1)
        cp_in[s].wait()
        if s >= 2:
            cp_out[s - 2].wait()
        compute(s)
        cp_out[s] = start_out(s)
    cp_out[NSLOTS - 2].wait()
    cp_out[NSLOTS - 1].wait()


def kernel(x, table, W, b):
    tab = jnp.concatenate([table.reshape(-1), jnp.zeros((11,), jnp.float32)])
    wb = jnp.concatenate(
        [jnp.zeros((1,), jnp.float32), W.reshape(-1), b.reshape(-1),
         jnp.zeros((11,), jnp.float32)])
    xf = (x.T.reshape(SEQ // 8, 8, BATCH // 128, 128)
          .transpose(0, 2, 1, 3).reshape(-1))
    out = _sc_lookup(xf, tab, wb)
    return out.transpose(1, 2, 0).reshape(BATCH, SEQ)[:, :, None]


# speedup vs baseline: 1.1073x; 1.0331x over previous
import functools

import jax
import jax.numpy as jnp
from jax import lax
from jax.experimental import pallas as pl
from jax.experimental.pallas import tpu as pltpu
from jax.experimental.pallas import tpu_sc as plsc

NC, NS, L = 2, 16, 16
NW = NC * NS
BATCH, SEQ = 16384, 200
N = BATCH * SEQ
CTILES = BATCH // 128
TROWS = SEQ // 8
TPB = 16
UNIT = TPB * 1024
NUNITS = TROWS * (CTILES // TPB)
FULL_PER_W = 6
QTILES = 4
QUNIT = QTILES * 1024

_mesh = plsc.VectorSubcoreMesh(
    core_axis_name="c", subcore_axis_name="s", num_cores=NC, num_subcores=NS)


@functools.partial(
    pl.kernel,
    out_type=jax.ShapeDtypeStruct((SEQ, CTILES, 128), jnp.float32),
    mesh=_mesh,
    compiler_params=pltpu.CompilerParams(needs_layout_passes=False),
    scratch_types=[
        pltpu.VMEM((32,), jnp.float32),
        pltpu.VMEM((16,), jnp.float32),
        pltpu.VMEM((16,), jnp.float32),
        pltpu.VMEM((UNIT,), jnp.int32),
        pltpu.VMEM((UNIT,), jnp.int32),
        pltpu.VMEM((UNIT,), jnp.int32),
        pltpu.VMEM((8, TPB, 128), jnp.float32),
        pltpu.VMEM((8, TPB, 128), jnp.float32),
        pltpu.VMEM((8, TPB, 128), jnp.float32),
        pltpu.SemaphoreType.DMA,
        pltpu.SemaphoreType.DMA,
        pltpu.SemaphoreType.DMA,
        pltpu.SemaphoreType.DMA,
        pltpu.SemaphoreType.DMA,
        pltpu.SemaphoreType.DMA,
    ],
)
def _sc_lookup(x_hbm, tab_hbm, wb_hbm, out_hbm, tab_v, wb_v, lut_v, xin0,
               xin1, xin2, yout0, yout1, yout2, in_sem0, in_sem1, in_sem2,
               out_sem0, out_sem1, out_sem2):
    wid = lax.axis_index("s") * NC + lax.axis_index("c")
    xin_b = (xin0, xin1, xin2)
    yout_b = (yout0, yout1, yout2)
    in_sems = (in_sem0, in_sem1, in_sem2)
    out_sems = (out_sem0, out_sem1, out_sem2)

    lane = lax.iota(jnp.int32, 16)
    rowmask = lane < 7
    zero16 = jnp.zeros((16,), jnp.int32)

    def col(j):
        idx = jnp.where(rowmask, lane * 3 + j, 0)
        return plsc.load_gather(tab_v, [idx])

    def scalar_bcast(j):
        return plsc.load_gather(wb_v, [zero16 + (j + 1)])

    def make_lut():
        pltpu.sync_copy(tab_hbm, tab_v)
        pltpu.sync_copy(wb_hbm, wb_v)
        z = (col(0) * scalar_bcast(0)
             + col(1) * scalar_bcast(1)
             + col(2) * scalar_bcast(2)
             + scalar_bcast(3))
        lutv = 1.0 / (1.0 + jnp.exp(-z))
        lut_v[...] = jnp.where(rowmask, lutv, 0.0)

    q = lax.rem(wid, 4)
    tail_m = lax.div(wid, 4)

    def unit_coords(slot):
        if slot < FULL_PER_W:
            u = wid + NW * slot
            tr = lax.div(u, 8)
            m = lax.rem(u, 8)
            words = UNIT
            in_off = (tr * CTILES + m * TPB) * 1024
        else:
            tr = TROWS - 1
            m = tail_m
            words = QUNIT
            in_off = (tr * CTILES + m * TPB) * 1024 + q * QUNIT
        return tr, m, words, in_off

    def start_in(slot):
        _, _, words, in_off = unit_coords(slot)
        off = pl.multiple_of(in_off, 8)
        return pltpu.async_copy(
            x_hbm.at[pl.ds(off, words)],
            xin_b[slot % 3].at[pl.ds(0, words)], in_sems[slot % 3])

    def start_out(slot):
        tr, m, words, _ = unit_coords(slot)
        yout = yout_b[slot % 3]
        ntiles = words // 1024
        t0 = m * TPB + (q * QTILES if slot >= FULL_PER_W else 0)
        return pltpu.async_copy(
            yout.at[pl.ds(0, 8), pl.ds(0, ntiles), pl.ds(0, 128)],
            out_hbm.at[pl.ds(8 * tr, 8),
                       pl.ds(pl.multiple_of(t0, 4), ntiles), pl.ds(0, 128)],
            out_sems[slot % 3])

    def compute(slot):
        xin = xin_b[slot % 3]
        yout = yout_b[slot % 3]
        ntiles = TPB if slot < FULL_PER_W else QTILES

        @plsc.parallel_loop(0, ntiles * 8, 1)
        def _(vv):
            t = lax.shift_right_logical(vv, 3)
            v = lax.bitwise_and(vv, 7)
            base_w = t * 1024 + v * 16
            for ri in range(8):
                yout[ri, t, pl.ds(v * 16, L)] = plsc.load_gather(
                    lut_v, [xin[pl.ds(base_w + ri * 128, L)]])

    NSLOTS = FULL_PER_W + 1
    cp_in = [None] * NSLOTS
    cp_out = [None] * NSLOTS
    cp_in[0] = start_in(0)
    cp_in[1] = start_in(1)
    make_lut()
    for s in range(NSLOTS):
        if s + 2 < NSLOTS:
            cp_in[s + 2] = start_in(s + 2)
        cp_in[s].wait()
        if s >= 3:
            cp_out[s - 3].wait()
        compute(s)
        cp_out[s] = start_out(s)
    cp_out[NSLOTS - 3].wait()
    cp_out[NSLOTS - 2].wait()
    cp_out[NSLOTS - 1].wait()


def kernel(x, table, W, b):
    tab = jnp.concatenate([table.reshape(-1), jnp.zeros((11,), jnp.float32)])
    wb = jnp.concatenate(
        [jnp.zeros((1,), jnp.float32), W.reshape(-1), b.reshape(-1),
         jnp.zeros((11,), jnp.float32)])
    xf = (x.T.reshape(SEQ // 8, 8, BATCH // 128, 128)
          .transpose(0, 2, 1, 3).reshape(-1))
    out = _sc_lookup(xf, tab, wb)
    return out.transpose(1, 2, 0).reshape(BATCH, SEQ)[:, :, None]
